# baseline (speedup 1.0000x reference)
import jax
import jax.numpy as jnp
from jax.experimental import pallas as pl
from jax.experimental.pallas import tpu as pltpu

_ROWS = 128


def _exchange_body(t_ref, w1_ref, w2_ref, x0_ref, x1_ref, y1_ref, y2_ref):
    t = t_ref[0]
    m1 = jnp.abs(w1_ref[...]) >= t
    m2 = jnp.abs(w2_ref[...]) >= t
    a0 = x0_ref[...]
    a1 = x1_ref[...]
    y1_ref[...] = jnp.where(m1, a0, a1)
    y2_ref[...] = jnp.where(m2, a1, a0)


def kernel(x0, x1, bn1_weight, bn2_weight, bn_threshold):
    B, C, H, W = x0.shape
    HW = H * W
    R = B * C
    x0f = x0.reshape(R, HW)
    x1f = x1.reshape(R, HW)
    w1 = jnp.tile(bn1_weight, B).reshape(R, 1)
    w2 = jnp.tile(bn2_weight, B).reshape(R, 1)
    t = bn_threshold.reshape(1)

    grid = (R // _ROWS,)
    y1, y2 = pl.pallas_call(
        _exchange_body,
        grid=grid,
        in_specs=[
            pl.BlockSpec(memory_space=pltpu.SMEM),
            pl.BlockSpec((_ROWS, 1), lambda i: (i, 0)),
            pl.BlockSpec((_ROWS, 1), lambda i: (i, 0)),
            pl.BlockSpec((_ROWS, HW), lambda i: (i, 0)),
            pl.BlockSpec((_ROWS, HW), lambda i: (i, 0)),
        ],
        out_specs=[
            pl.BlockSpec((_ROWS, HW), lambda i: (i, 0)),
            pl.BlockSpec((_ROWS, HW), lambda i: (i, 0)),
        ],
        out_shape=[jax.ShapeDtypeStruct((R, HW), x0.dtype)] * 2,
    )(t, w1, w2, x0f, x1f)
    return y1.reshape(B, C, H, W), y2.reshape(B, C, H, W)


# speedup vs baseline: 1.0399x; 1.0399x over previous
import jax
import jax.numpy as jnp
from jax.experimental import pallas as pl
from jax.experimental.pallas import tpu as pltpu

_CBLK = 16


def _exchange_body(w1_ref, w2_ref, t_ref, x0_ref, x1_ref, y1_ref, y2_ref):
    c0 = pl.program_id(1) * _CBLK
    t = t_ref[0]
    for k in range(_CBLK):
        p1 = jnp.abs(w1_ref[c0 + k]) >= t
        p2 = jnp.abs(w2_ref[c0 + k]) >= t
        a0 = x0_ref[0, k]
        a1 = x1_ref[0, k]
        y1_ref[0, k] = jnp.where(p1, a0, a1)
        y2_ref[0, k] = jnp.where(p2, a1, a0)


def kernel(x0, x1, bn1_weight, bn2_weight, bn_threshold):
    B, C, H, W = x0.shape
    t = bn_threshold.reshape(1)

    grid = (B, C // _CBLK)
    blk = pl.BlockSpec((1, _CBLK, H, W), lambda b, c: (b, c, 0, 0))
    y1, y2 = pl.pallas_call(
        _exchange_body,
        grid=grid,
        in_specs=[
            pl.BlockSpec(memory_space=pltpu.SMEM),
            pl.BlockSpec(memory_space=pltpu.SMEM),
            pl.BlockSpec(memory_space=pltpu.SMEM),
            blk,
            blk,
        ],
        out_specs=[blk, blk],
        out_shape=[jax.ShapeDtypeStruct((B, C, H, W), x0.dtype)] * 2,
    )(bn1_weight, bn2_weight, t, x0, x1)
    return y1, y2


# speedup vs baseline: 6.5111x; 6.2611x over previous
import jax
import jax.numpy as jnp
from jax.experimental import pallas as pl
from jax.experimental.pallas import tpu as pltpu

_HBLK = 8


def _exchange_body(w1_ref, w2_ref, t_ref, x0_ref, x1_ref, y1_ref, y2_ref):
    t = t_ref[0]
    m1 = jnp.abs(w1_ref[...]) >= t
    m2 = jnp.abs(w2_ref[...]) >= t
    a0 = x0_ref[...]
    a1 = x1_ref[...]
    y1_ref[...] = jnp.where(m1[None, None], a0, a1)
    y2_ref[...] = jnp.where(m2[None, None], a1, a0)


def kernel(x0, x1, bn1_weight, bn2_weight, bn_threshold):
    B, C, H, W = x0.shape
    x0t = jnp.transpose(x0, (0, 2, 3, 1))
    x1t = jnp.transpose(x1, (0, 2, 3, 1))
    w1 = bn1_weight.reshape(1, C)
    w2 = bn2_weight.reshape(1, C)
    t = bn_threshold.reshape(1)

    grid = (B, H // _HBLK)
    blk = pl.BlockSpec((1, _HBLK, W, C), lambda b, h: (b, h, 0, 0))
    wblk = pl.BlockSpec((1, C), lambda b, h: (0, 0))
    y1t, y2t = pl.pallas_call(
        _exchange_body,
        grid=grid,
        in_specs=[
            wblk,
            wblk,
            pl.BlockSpec(memory_space=pltpu.SMEM),
            blk,
            blk,
        ],
        out_specs=[blk, blk],
        out_shape=[jax.ShapeDtypeStruct((B, H, W, C), x0.dtype)] * 2,
    )(w1, w2, t, x0t, x1t)
    return (jnp.transpose(y1t, (0, 3, 1, 2)),
            jnp.transpose(y2t, (0, 3, 1, 2)))


# speedup vs baseline: 7.4943x; 1.1510x over previous
import jax
import jax.numpy as jnp
from jax.experimental import pallas as pl
from jax.experimental.pallas import tpu as pltpu

_HBLK = 16


def _exchange_body(w1_ref, w2_ref, t_ref, x0_ref, x1_ref, y1_ref, y2_ref):
    t = t_ref[0]
    m1 = jnp.abs(w1_ref[...]) >= t
    m2 = jnp.abs(w2_ref[...]) >= t
    a0 = x0_ref[...]
    a1 = x1_ref[...]
    y1_ref[...] = jnp.where(m1[None, None], a0, a1)
    y2_ref[...] = jnp.where(m2[None, None], a1, a0)


def kernel(x0, x1, bn1_weight, bn2_weight, bn_threshold):
    B, C, H, W = x0.shape
    x0t = jnp.transpose(x0, (0, 2, 3, 1))
    x1t = jnp.transpose(x1, (0, 2, 3, 1))
    w1 = bn1_weight.reshape(1, C)
    w2 = bn2_weight.reshape(1, C)
    t = bn_threshold.reshape(1)

    grid = (B, H // _HBLK)
    blk = pl.BlockSpec((1, _HBLK, W, C), lambda b, h: (b, h, 0, 0))
    wblk = pl.BlockSpec((1, C), lambda b, h: (0, 0))
    y1t, y2t = pl.pallas_call(
        _exchange_body,
        grid=grid,
        in_specs=[
            wblk,
            wblk,
            pl.BlockSpec(memory_space=pltpu.SMEM),
            blk,
            blk,
        ],
        out_specs=[blk, blk],
        out_shape=[jax.ShapeDtypeStruct((B, H, W, C), x0.dtype)] * 2,
    )(w1, w2, t, x0t, x1t)
    return (jnp.transpose(y1t, (0, 3, 1, 2)),
            jnp.transpose(y2t, (0, 3, 1, 2)))


# speedup vs baseline: 7.7391x; 1.0327x over previous
import jax
import jax.numpy as jnp
from jax.experimental import pallas as pl
from jax.experimental.pallas import tpu as pltpu

_HBLK = 32


def _exchange_body(w1_ref, w2_ref, t_ref, x0_ref, x1_ref, y1_ref, y2_ref):
    t = t_ref[0]
    m1 = jnp.abs(w1_ref[...]) >= t
    m2 = jnp.abs(w2_ref[...]) >= t
    a0 = x0_ref[...]
    a1 = x1_ref[...]
    y1_ref[...] = jnp.where(m1[None, None], a0, a1)
    y2_ref[...] = jnp.where(m2[None, None], a1, a0)


def kernel(x0, x1, bn1_weight, bn2_weight, bn_threshold):
    B, C, H, W = x0.shape
    x0t = jnp.transpose(x0, (0, 2, 3, 1))
    x1t = jnp.transpose(x1, (0, 2, 3, 1))
    w1 = bn1_weight.reshape(1, C)
    w2 = bn2_weight.reshape(1, C)
    t = bn_threshold.reshape(1)

    grid = (B, H // _HBLK)
    blk = pl.BlockSpec((1, _HBLK, W, C), lambda b, h: (b, h, 0, 0))
    wblk = pl.BlockSpec((1, C), lambda b, h: (0, 0))
    y1t, y2t = pl.pallas_call(
        _exchange_body,
        grid=grid,
        in_specs=[
            wblk,
            wblk,
            pl.BlockSpec(memory_space=pltpu.SMEM),
            blk,
            blk,
        ],
        out_specs=[blk, blk],
        out_shape=[jax.ShapeDtypeStruct((B, H, W, C), x0.dtype)] * 2,
    )(w1, w2, t, x0t, x1t)
    return (jnp.transpose(y1t, (0, 3, 1, 2)),
            jnp.transpose(y2t, (0, 3, 1, 2)))


# speedup vs baseline: 8.1538x; 1.0536x over previous
import jax
import jax.numpy as jnp
from jax.experimental import pallas as pl
from jax.experimental.pallas import tpu as pltpu

_HBLK = 64


def _exchange_body(w1_ref, w2_ref, t_ref, x0_ref, x1_ref, y1_ref, y2_ref):
    t = t_ref[0]
    m1 = jnp.abs(w1_ref[...]) >= t
    m2 = jnp.abs(w2_ref[...]) >= t
    a0 = x0_ref[...]
    a1 = x1_ref[...]
    y1_ref[...] = jnp.where(m1[None, None], a0, a1)
    y2_ref[...] = jnp.where(m2[None, None], a1, a0)


def kernel(x0, x1, bn1_weight, bn2_weight, bn_threshold):
    B, C, H, W = x0.shape
    x0t = jnp.transpose(x0, (0, 2, 3, 1))
    x1t = jnp.transpose(x1, (0, 2, 3, 1))
    w1 = bn1_weight.reshape(1, C)
    w2 = bn2_weight.reshape(1, C)
    t = bn_threshold.reshape(1)

    grid = (B, H // _HBLK)
    blk = pl.BlockSpec((1, _HBLK, W, C), lambda b, h: (b, h, 0, 0))
    wblk = pl.BlockSpec((1, C), lambda b, h: (0, 0))
    y1t, y2t = pl.pallas_call(
        _exchange_body,
        grid=grid,
        in_specs=[
            wblk,
            wblk,
            pl.BlockSpec(memory_space=pltpu.SMEM),
            blk,
            blk,
        ],
        out_specs=[blk, blk],
        out_shape=[jax.ShapeDtypeStruct((B, H, W, C), x0.dtype)] * 2,
    )(w1, w2, t, x0t, x1t)
    return (jnp.transpose(y1t, (0, 3, 1, 2)),
            jnp.transpose(y2t, (0, 3, 1, 2)))
